# baseline (speedup 1.0000x reference)
import jax
import jax.numpy as jnp
from jax import lax
from jax.experimental import pallas as pl
from jax.experimental.pallas import tpu as pltpu
from jax.experimental.pallas import tpu_sc as plsc

B = 16384
D = 64
NC = 2
NS = 16
NW = NC * NS
BPW = B // NW
C = 32
NCH = BPW // C
NT = 11


def _sc_body(obs_0, obs_1, obs_2, obs_3, obs_4, obs_5, obs_6, obs_7,
             prev_action,
             w_state_0, w_state_1, w_state_2, w_state_3,
             w_state_4, w_state_5, w_state_6, w_state_7,
             w_act_0, w_act_1, w_act_2, bias,
             out,
             idx_buf, rows, out_buf, bias_buf,
             sem_g0, sem_g1, sem_o0, sem_o1):
    obs = (obs_0, obs_1, obs_2, obs_3, obs_4, obs_5, obs_6, obs_7)
    tables = (w_state_0, w_state_1, w_state_2, w_state_3,
              w_state_4, w_state_5, w_state_6, w_state_7,
              w_act_0, w_act_1, w_act_2)
    sem_g = (sem_g0, sem_g1)
    sem_o = (sem_o0, sem_o1)

    wid = lax.axis_index("s") * NC + lax.axis_index("c")
    obase = wid * BPW
    pbase = wid * (BPW // 2)

    pltpu.sync_copy(bias, bias_buf)
    for t in range(8):
        pltpu.sync_copy(obs[t].at[pl.ds(obase, BPW)], idx_buf.at[t])
    pltpu.sync_copy(prev_action.at[pl.ds(obase, BPW)], idx_buf.at[8])

    ten = jnp.full((16,), 10, jnp.int32)
    tenth = jnp.full((16,), 0.1, jnp.float32)
    one = jnp.full((16,), 1, jnp.int32)
    for j in range(BPW // 16):
        sl = pl.ds(j * 16, 16)
        v = idx_buf[8, sl]
        q1 = (v.astype(jnp.float32) * tenth).astype(jnp.int32)
        q2 = (q1.astype(jnp.float32) * tenth).astype(jnp.int32)
        q3 = (q2.astype(jnp.float32) * tenth).astype(jnp.int32)
        idx_buf[8, sl] = v - q1 * ten
        idx_buf[9, sl] = q1 - q2 * ten
        idx_buf[10, sl] = q2 - q3 * ten
    for t in range(NT):
        for j in range(BPW // 16):
            sl = pl.ds(j * 16, 16)
            idx_buf[t, sl] = lax.shift_right_logical(idx_buf[t, sl], one)

    def fire(g, b):
        for t in range(NT):
            pltpu.async_copy(
                tables[t].at[idx_buf.at[t, pl.ds(g * C, C)]],
                rows.at[b, t], sem_g[b])

    def wait_gathers(g, b):
        for t in range(NT):
            pltpu.make_async_copy(
                tables[t].at[idx_buf.at[t, pl.ds(g * C, C)]],
                rows.at[b, t], sem_g[b]).wait()

    fire(0, 0)

    def outer(i, carry):
        for b in range(2):
            g = 2 * i + b

            @pl.when(g + 1 < NCH)
            def _():
                fire(g + 1, 1 - b)

            wait_gathers(g, b)

            @pl.when(g >= 2)
            def _():
                pltpu.make_async_copy(
                    out_buf.at[b],
                    out.at[pl.ds(pbase + (g - 2) * (C // 2), C // 2)],
                    sem_o[b]).wait()

            def row_body(r, rcarry):
                for kcol in range(8):
                    sl = pl.ds(kcol * 16, 16)
                    acc = bias_buf[pl.ds((kcol % 4) * 16, 16)]
                    for t in range(NT):
                        acc = acc + rows[b, t, r, sl]
                    out_buf[b, lax.div(r, 2), sl] = acc
                return rcarry

            lax.fori_loop(0, C, row_body, 0)
            pltpu.async_copy(
                out_buf.at[b],
                out.at[pl.ds(pbase + g * (C // 2), C // 2)], sem_o[b])
        return carry

    lax.fori_loop(0, NCH // 2, outer, 0)

    for b in range(2):
        g = NCH - 2 + b
        pltpu.make_async_copy(
            out_buf.at[b],
            out.at[pl.ds(pbase + g * (C // 2), C // 2)], sem_o[b]).wait()


def kernel(obs_0, obs_1, obs_2, obs_3, obs_4, obs_5, obs_6, obs_7,
           prev_action,
           w_state_0, w_state_1, w_state_2, w_state_3,
           w_state_4, w_state_5, w_state_6, w_state_7,
           w_act_0, w_act_1, w_act_2, bias):
    ws = [w.reshape(-1, 128) for w in
          (w_state_0, w_state_1, w_state_2, w_state_3,
           w_state_4, w_state_5, w_state_6, w_state_7,
           w_act_0, w_act_1, w_act_2)]
    mesh = plsc.VectorSubcoreMesh(core_axis_name="c", subcore_axis_name="s")
    run = pl.kernel(
        _sc_body,
        out_type=jax.ShapeDtypeStruct((B // 2, 128), jnp.float32),
        mesh=mesh,
        scratch_types=[
            pltpu.VMEM((NT, BPW), jnp.int32),
            pltpu.VMEM((2, NT, C, 128), jnp.float32),
            pltpu.VMEM((2, C // 2, 128), jnp.float32),
            pltpu.VMEM((D,), jnp.float32),
            pltpu.SemaphoreType.DMA,
            pltpu.SemaphoreType.DMA,
            pltpu.SemaphoreType.DMA,
            pltpu.SemaphoreType.DMA,
        ],
        compiler_params=pltpu.CompilerParams(use_tc_tiling_on_sc=True),
    )
    out = run(obs_0, obs_1, obs_2, obs_3, obs_4, obs_5, obs_6, obs_7,
              prev_action, *ws, bias)
    return out.reshape(B, D)


# speedup vs baseline: 1.1119x; 1.1119x over previous
import jax
import jax.numpy as jnp
from jax import lax
from jax.experimental import pallas as pl
from jax.experimental.pallas import tpu as pltpu
from jax.experimental.pallas import tpu_sc as plsc

B = 16384
D = 64
NC = 2
NS = 16
NW = NC * NS
BPW = B // NW
C = 128
NCH = BPW // C
NBIG = 2


def _make_body(mode):
    nact = 3 if mode == "last" else 0
    ngather = NBIG + nact
    nsrc = ngather + (0 if mode == "first" else 1)
    nidx = ngather if mode != "last" else ngather + 0

    def body(*refs):
        if mode == "last":
            (obs_a, obs_b, prev_action, w_a, w_b,
             w_act_0, w_act_1, w_act_2, part,
             out, idx_buf, rows, out_buf, bias_buf,
             sem_g0, sem_g1, sem_o0, sem_o1) = refs
            tables = (w_a, w_b, w_act_0, w_act_1, w_act_2)
        elif mode == "mid":
            (obs_a, obs_b, w_a, w_b, part,
             out, idx_buf, rows, out_buf, bias_buf,
             sem_g0, sem_g1, sem_o0, sem_o1) = refs
            tables = (w_a, w_b)
        else:
            (obs_a, obs_b, w_a, w_b, bias,
             out, idx_buf, rows, out_buf, bias_buf,
             sem_g0, sem_g1, sem_o0, sem_o1) = refs
            tables = (w_a, w_b)
        obs = (obs_a, obs_b)
        sem_g = (sem_g0, sem_g1)
        sem_o = (sem_o0, sem_o1)

        wid = lax.axis_index("s") * NC + lax.axis_index("c")
        obase = wid * BPW

        for t in range(NBIG):
            pltpu.sync_copy(obs[t].at[pl.ds(obase, BPW)], idx_buf.at[t])
        if mode == "last":
            pltpu.sync_copy(prev_action.at[pl.ds(obase, BPW)],
                            idx_buf.at[NBIG])
            ten = jnp.full((16,), 10, jnp.int32)
            tenth = jnp.full((16,), 0.1, jnp.float32)
            for j in range(BPW // 16):
                sl = pl.ds(j * 16, 16)
                v = idx_buf[NBIG, sl]
                q1 = (v.astype(jnp.float32) * tenth).astype(jnp.int32)
                q2 = (q1.astype(jnp.float32) * tenth).astype(jnp.int32)
                q3 = (q2.astype(jnp.float32) * tenth).astype(jnp.int32)
                idx_buf[NBIG, sl] = v - q1 * ten
                idx_buf[NBIG + 1, sl] = q1 - q2 * ten
                idx_buf[NBIG + 2, sl] = q2 - q3 * ten
        if mode == "first":
            pltpu.sync_copy(bias, bias_buf)

        def fire(g, b):
            for t in range(ngather):
                pltpu.async_copy(
                    tables[t].at[idx_buf.at[t, pl.ds(g * C, C)]],
                    rows.at[b, t], sem_g[b])
            if mode != "first":
                pltpu.async_copy(part.at[pl.ds(obase + g * C, C)],
                                 rows.at[b, nsrc - 1], sem_g[b])

        def wait_gathers(g, b):
            for t in range(ngather):
                pltpu.make_async_copy(
                    tables[t].at[idx_buf.at[t, pl.ds(g * C, C)]],
                    rows.at[b, t], sem_g[b]).wait()
            if mode != "first":
                pltpu.make_async_copy(part.at[pl.ds(obase + g * C, C)],
                                      rows.at[b, nsrc - 1], sem_g[b]).wait()

        fire(0, 0)

        def outer(i, carry):
            for b in range(2):
                g = 2 * i + b

                @pl.when(g + 1 < NCH)
                def _():
                    fire(g + 1, 1 - b)

                wait_gathers(g, b)

                @pl.when(g >= 2)
                def _():
                    pltpu.make_async_copy(
                        out_buf.at[b],
                        out.at[pl.ds(obase + (g - 2) * C, C)],
                        sem_o[b]).wait()

                def row_body(r, rcarry):
                    for kcol in range(D // 16):
                        sl = pl.ds(kcol * 16, 16)
                        if mode == "first":
                            acc = bias_buf[sl]
                        else:
                            acc = rows[b, nsrc - 1, r, sl]
                        for t in range(ngather):
                            acc = acc + rows[b, t, r, sl]
                        out_buf[b, r, sl] = acc
                    return rcarry

                lax.fori_loop(0, C, row_body, 0)
                pltpu.async_copy(
                    out_buf.at[b], out.at[pl.ds(obase + g * C, C)],
                    sem_o[b])
            return carry

        lax.fori_loop(0, NCH // 2, outer, 0)

        for b in range(2):
            g = NCH - 2 + b
            pltpu.make_async_copy(
                out_buf.at[b], out.at[pl.ds(obase + g * C, C)],
                sem_o[b]).wait()

    return body, nsrc, max(nidx, NBIG + nact)


def _group_kernel(mode):
    body, nsrc, nidx = _make_body(mode)
    return pl.kernel(
        body,
        out_type=jax.ShapeDtypeStruct((B, D), jnp.float32),
        mesh=plsc.VectorSubcoreMesh(core_axis_name="c", subcore_axis_name="s"),
        scratch_types=[
            pltpu.VMEM((nidx, BPW), jnp.int32),
            pltpu.VMEM((2, nsrc, C, D), jnp.float32),
            pltpu.VMEM((2, C, D), jnp.float32),
            pltpu.VMEM((D,), jnp.float32),
            pltpu.SemaphoreType.DMA,
            pltpu.SemaphoreType.DMA,
            pltpu.SemaphoreType.DMA,
            pltpu.SemaphoreType.DMA,
        ],
        compiler_params=pltpu.CompilerParams(use_tc_tiling_on_sc=False),
    )


def kernel(obs_0, obs_1, obs_2, obs_3, obs_4, obs_5, obs_6, obs_7,
           prev_action,
           w_state_0, w_state_1, w_state_2, w_state_3,
           w_state_4, w_state_5, w_state_6, w_state_7,
           w_act_0, w_act_1, w_act_2, bias):
    first = _group_kernel("first")
    mid = _group_kernel("mid")
    fin = _group_kernel("last")

    part = first(obs_0, obs_1, w_state_0, w_state_1, bias)
    part = mid(obs_2, obs_3, w_state_2, w_state_3, part)
    part = mid(obs_4, obs_5, w_state_4, w_state_5, part)
    return fin(obs_6, obs_7, prev_action, w_state_6, w_state_7,
               w_act_0, w_act_1, w_act_2, part)


# speedup vs baseline: 1.1647x; 1.0475x over previous
import jax
import jax.numpy as jnp
from jax import lax
from jax.experimental import pallas as pl
from jax.experimental.pallas import tpu as pltpu
from jax.experimental.pallas import tpu_sc as plsc

B = 16384
D = 64
NC = 2
NS = 16
NW = NC * NS
BPW = B // NW
C = 128
NCH = BPW // C
NBIG = 2


def _make_body(mode):
    nact = 3 if mode == "first" else 0
    ngather = NBIG + nact
    nsrc = ngather + (1 if mode == "mid" else 0)
    nidx = ngather

    def body(*refs):
        if mode == "first":
            (obs_a, obs_b, prev_action, w_a, w_b,
             w_act_0, w_act_1, w_act_2, bias,
             out, idx_buf, rows, out_buf, bias_buf,
             sem_g0, sem_g1, sem_o0, sem_o1) = refs
            tables = (w_a, w_b, w_act_0, w_act_1, w_act_2)
        else:
            (obs_a, obs_b, w_a, w_b, part,
             out, idx_buf, rows, out_buf, bias_buf,
             sem_g0, sem_g1, sem_o0, sem_o1) = refs
            tables = (w_a, w_b)
        obs = (obs_a, obs_b)
        sem_g = (sem_g0, sem_g1)
        sem_o = (sem_o0, sem_o1)

        wid = lax.axis_index("s") * NC + lax.axis_index("c")
        obase = wid * BPW

        for t in range(NBIG):
            pltpu.sync_copy(obs[t].at[pl.ds(obase, BPW)], idx_buf.at[t])
        if mode == "first":
            pltpu.sync_copy(bias, bias_buf)
            pltpu.sync_copy(prev_action.at[pl.ds(obase, BPW)],
                            idx_buf.at[NBIG])
            ten = jnp.full((16,), 10, jnp.int32)
            tenth = jnp.full((16,), 0.1, jnp.float32)
            for j in range(BPW // 16):
                sl = pl.ds(j * 16, 16)
                v = idx_buf[NBIG, sl]
                q1 = (v.astype(jnp.float32) * tenth).astype(jnp.int32)
                q2 = (q1.astype(jnp.float32) * tenth).astype(jnp.int32)
                q3 = (q2.astype(jnp.float32) * tenth).astype(jnp.int32)
                idx_buf[NBIG, sl] = v - q1 * ten
                idx_buf[NBIG + 1, sl] = q1 - q2 * ten
                idx_buf[NBIG + 2, sl] = q2 - q3 * ten

        def fire(g, b):
            for t in range(ngather):
                pltpu.async_copy(
                    tables[t].at[idx_buf.at[t, pl.ds(g * C, C)]],
                    rows.at[b, t], sem_g[b])
            if mode == "mid":
                pltpu.async_copy(part.at[pl.ds(obase + g * C, C)],
                                 rows.at[b, nsrc - 1], sem_g[b])

        def wait_gathers(g, b):
            for t in range(ngather):
                pltpu.make_async_copy(
                    tables[t].at[idx_buf.at[t, pl.ds(g * C, C)]],
                    rows.at[b, t], sem_g[b]).wait()
            if mode == "mid":
                pltpu.make_async_copy(part.at[pl.ds(obase + g * C, C)],
                                      rows.at[b, nsrc - 1], sem_g[b]).wait()

        fire(0, 0)

        def outer(i, carry):
            for b in range(2):
                g = 2 * i + b

                @pl.when(g + 1 < NCH)
                def _():
                    fire(g + 1, 1 - b)

                wait_gathers(g, b)

                @pl.when(g >= 2)
                def _():
                    pltpu.make_async_copy(
                        out_buf.at[b],
                        out.at[pl.ds(obase + (g - 2) * C, C)],
                        sem_o[b]).wait()

                def row_body(r, rcarry):
                    for kcol in range(D // 16):
                        sl = pl.ds(kcol * 16, 16)
                        if mode == "first":
                            acc = bias_buf[sl]
                        else:
                            acc = rows[b, nsrc - 1, r, sl]
                        for t in range(ngather):
                            acc = acc + rows[b, t, r, sl]
                        out_buf[b, r, sl] = acc
                    return rcarry

                lax.fori_loop(0, C, row_body, 0)
                pltpu.async_copy(
                    out_buf.at[b], out.at[pl.ds(obase + g * C, C)],
                    sem_o[b])
            return carry

        lax.fori_loop(0, NCH // 2, outer, 0)

        for b in range(2):
            g = NCH - 2 + b
            pltpu.make_async_copy(
                out_buf.at[b], out.at[pl.ds(obase + g * C, C)],
                sem_o[b]).wait()

    return body, nsrc, nidx


def _group_kernel(mode):
    body, nsrc, nidx = _make_body(mode)
    return pl.kernel(
        body,
        out_type=jax.ShapeDtypeStruct((B, D), jnp.float32),
        mesh=plsc.VectorSubcoreMesh(core_axis_name="c", subcore_axis_name="s"),
        scratch_types=[
            pltpu.VMEM((nidx, BPW), jnp.int32),
            pltpu.VMEM((2, nsrc, C, D), jnp.float32),
            pltpu.VMEM((2, C, D), jnp.float32),
            pltpu.VMEM((D,), jnp.float32),
            pltpu.SemaphoreType.DMA,
            pltpu.SemaphoreType.DMA,
            pltpu.SemaphoreType.DMA,
            pltpu.SemaphoreType.DMA,
        ],
        compiler_params=pltpu.CompilerParams(use_tc_tiling_on_sc=False),
    )


def kernel(obs_0, obs_1, obs_2, obs_3, obs_4, obs_5, obs_6, obs_7,
           prev_action,
           w_state_0, w_state_1, w_state_2, w_state_3,
           w_state_4, w_state_5, w_state_6, w_state_7,
           w_act_0, w_act_1, w_act_2, bias):
    first = _group_kernel("first")
    mid = _group_kernel("mid")

    part = first(obs_0, obs_1, prev_action, w_state_0, w_state_1,
                 w_act_0, w_act_1, w_act_2, bias)
    part = mid(obs_2, obs_3, w_state_2, w_state_3, part)
    part = mid(obs_4, obs_5, w_state_4, w_state_5, part)
    return mid(obs_6, obs_7, w_state_6, w_state_7, part)
